# baseline (speedup 1.0000x reference)
import functools

import jax
import jax.numpy as jnp
from jax import lax
from jax.experimental import pallas as pl
from jax.experimental.pallas import tpu as pltpu
from jax.experimental.pallas import tpu_sc as plsc

N = 32768
B = 16
D = 1024
EPS = 1e-05

S_ROWS = 2048
ROWS = 2048

N_TC = 24576
N_SC = N - N_TC
NC = 2
NW = 32
W_ROWS = N_SC // NW
CH = 16
LANES = 16


def _onehot(starts_ref, ends_ref, step, rows):
    r = step * rows + jax.lax.broadcasted_iota(jnp.int32, (rows, B), 0)
    return ((r >= starts_ref[...]) & (r < ends_ref[...])).astype(jnp.float32)


def _stats_kernel(x_ref, starts_ref, ends_ref, sum_ref, sq_ref):
    step = pl.program_id(0)
    oh = _onehot(starts_ref, ends_ref, step, S_ROWS)
    x = x_ref[...]
    dims = (((0,), (0,)), ((), ()))
    ps = jax.lax.dot_general(oh, x, dims, preferred_element_type=jnp.float32)
    psq = jax.lax.dot_general(oh, x * x, dims,
                              preferred_element_type=jnp.float32)

    @pl.when(step == 0)
    def _():
        sum_ref[...] = ps
        sq_ref[...] = psq

    @pl.when(step != 0)
    def _():
        sum_ref[...] += ps
        sq_ref[...] += psq


def _sc_stats_body(x_hbm, off_hbm, zeros_hbm, sum_out, sq_out,
                   buf0, buf1, off_v, acc_s, acc_q, sem0, sem1):
    c = lax.axis_index("c")
    s = lax.axis_index("s")
    wid = s * NC + c
    base = N_TC + wid * W_ROWS

    pltpu.sync_copy(zeros_hbm, acc_s)
    pltpu.sync_copy(zeros_hbm, acc_q)
    pltpu.sync_copy(off_hbm, off_v)

    off_vec = off_v[...]
    lane = lax.iota(jnp.int32, LANES)

    def start(ci, buf, sem):
        row0 = jnp.minimum(base + ci * CH, N - CH)
        pltpu.make_async_copy(x_hbm.at[pl.ds(row0, CH)], buf, sem).start()

    def wait(buf, sem):
        pltpu.make_async_copy(x_hbm.at[pl.ds(0, CH)], buf, sem).wait()

    def compute(ci, buf):
        row0 = base + ci * CH

        def row_fn(j, _):
            seg = plsc.all_reduce_population_count(
                off_vec <= jnp.full((LANES,), row0 + j, jnp.int32))

            for k in range(D // LANES):
                col = k * LANES
                v = buf[j, pl.ds(col, LANES)]
                plsc.addupdate_scatter(acc_s, [seg, col + lane], v)
                plsc.addupdate_scatter(acc_q, [seg, col + lane], v * v)
            return 0
        lax.fori_loop(0, CH, row_fn, 0)

    start(0, buf0, sem0)

    def pair(p, _):
        ci = 2 * p
        start(ci + 1, buf1, sem1)
        wait(buf0, sem0)
        compute(ci, buf0)
        start(ci + 2, buf0, sem0)
        wait(buf1, sem1)
        compute(ci + 1, buf1)
        return 0

    lax.fori_loop(0, W_ROWS // CH // 2, pair, 0)
    wait(buf0, sem0)

    pltpu.sync_copy(acc_s, sum_out.at[pl.ds(wid * B, B)])
    pltpu.sync_copy(acc_q, sq_out.at[pl.ds(wid * B, B)])


def _sc_stats(x, offsets, interpret=False):
    zeros = jnp.zeros((B, D), jnp.float32)
    fn = pl.kernel(
        _sc_stats_body,
        out_type=[jax.ShapeDtypeStruct((NW * B, D), jnp.float32),
                  jax.ShapeDtypeStruct((NW * B, D), jnp.float32)],
        mesh=plsc.VectorSubcoreMesh(core_axis_name="c",
                                    subcore_axis_name="s",
                                    num_cores=NC, num_subcores=NW // NC),
        scratch_types=[pltpu.VMEM((CH, D), jnp.float32),
                       pltpu.VMEM((CH, D), jnp.float32),
                       pltpu.VMEM((LANES,), jnp.int32),
                       pltpu.VMEM((B, D), jnp.float32),
                       pltpu.VMEM((B, D), jnp.float32),
                       pltpu.SemaphoreType.DMA,
                       pltpu.SemaphoreType.DMA],
        compiler_params=pltpu.CompilerParams(needs_layout_passes=False),
        interpret=interpret,
    )
    return fn(x, offsets, zeros)


def _prep_kernel(sum_ref, sq_ref, scs_ref, scq_ref, w_ref, b_ref, invc_ref,
                 scale_ref, shift_ref):
    inv = invc_ref[:, 0:1]
    s = sum_ref[...]
    sq = sq_ref[...]
    for w in range(NW):
        s = s + scs_ref[w * B:(w + 1) * B, :]
        sq = sq + scq_ref[w * B:(w + 1) * B, :]
    mean = s * inv
    var = sq * inv - mean * mean
    rstd = jax.lax.rsqrt(jnp.maximum(var, 0.0) + EPS)
    scale = rstd * w_ref[...]
    scale_ref[...] = scale
    shift_ref[...] = b_ref[...] - mean * scale


def _norm_kernel(x_ref, scale_ref, shift_ref, starts_ref, ends_ref, out_ref):
    step = pl.program_id(0)
    oh = _onehot(starts_ref, ends_ref, step, ROWS)
    dims = (((1,), (0,)), ((), ()))
    row_scale = jax.lax.dot_general(oh, scale_ref[...], dims,
                                    preferred_element_type=jnp.float32)
    row_shift = jax.lax.dot_general(oh, shift_ref[...], dims,
                                    preferred_element_type=jnp.float32)
    out_ref[...] = x_ref[...] * row_scale + row_shift


@functools.partial(jax.jit, static_argnames=("interpret",))
def kernel(input, offsets, weight, bias, interpret=False):
    s_steps = N_TC // S_ROWS
    steps = N // ROWS
    ends = offsets.reshape(1, B)
    starts = jnp.concatenate(
        [jnp.zeros((1, 1), jnp.int32), ends[:, :-1]], axis=1)
    invc = jnp.broadcast_to(
        (1.0 / jnp.maximum(ends - starts, 1).astype(jnp.float32)).reshape(
            B, 1), (B, 128))

    scs, scq = _sc_stats(input, offsets, interpret=interpret)

    small = pl.BlockSpec((1, B), lambda i: (0, 0))
    ssum, ssq = pl.pallas_call(
        _stats_kernel,
        grid=(s_steps,),
        in_specs=[pl.BlockSpec((S_ROWS, D), lambda i: (i, 0)), small, small],
        out_specs=[pl.BlockSpec((B, D), lambda i: (0, 0)),
                   pl.BlockSpec((B, D), lambda i: (0, 0))],
        out_shape=[jax.ShapeDtypeStruct((B, D), jnp.float32),
                   jax.ShapeDtypeStruct((B, D), jnp.float32)],
        interpret=interpret,
    )(input, starts, ends)

    scale, shift = pl.pallas_call(
        _prep_kernel,
        out_shape=[jax.ShapeDtypeStruct((B, D), jnp.float32),
                   jax.ShapeDtypeStruct((B, D), jnp.float32)],
        interpret=interpret,
    )(ssum, ssq, scs, scq, weight.reshape(1, D), bias.reshape(1, D), invc)

    out = pl.pallas_call(
        _norm_kernel,
        grid=(steps,),
        in_specs=[pl.BlockSpec((ROWS, D), lambda i: (i, 0)),
                  pl.BlockSpec((B, D), lambda i: (0, 0)),
                  pl.BlockSpec((B, D), lambda i: (0, 0)),
                  small, small],
        out_specs=pl.BlockSpec((ROWS, D), lambda i: (i, 0)),
        out_shape=jax.ShapeDtypeStruct((N, D), jnp.float32),
        interpret=interpret,
    )(input, scale, shift, starts, ends)
    return out


# speedup vs baseline: 1.1249x; 1.1249x over previous
import functools

import jax
import jax.numpy as jnp
from jax import lax
from jax.experimental import pallas as pl
from jax.experimental.pallas import tpu as pltpu
from jax.experimental.pallas import tpu_sc as plsc

N = 32768
B = 16
D = 1024
EPS = 1e-05

S_ROWS = 2048
ROWS = 2048

N_TC = 24576
N_SC = N - N_TC
NC = 2
NW = 32
W_ROWS = N_SC // NW
CH = 16
LANES = 16


def _onehot(starts_ref, ends_ref, step, rows):
    r = step * rows + jax.lax.broadcasted_iota(jnp.int32, (rows, B), 0)
    return ((r >= starts_ref[...]) & (r < ends_ref[...])).astype(jnp.float32)


def _stats_kernel(x_ref, starts_ref, ends_ref, sum_ref, sq_ref):
    step = pl.program_id(0)
    oh = _onehot(starts_ref, ends_ref, step, S_ROWS)
    x = x_ref[...]
    dims = (((0,), (0,)), ((), ()))
    ps = jax.lax.dot_general(oh, x, dims, preferred_element_type=jnp.float32)
    psq = jax.lax.dot_general(oh, x * x, dims,
                              preferred_element_type=jnp.float32)

    @pl.when(step == 0)
    def _():
        sum_ref[...] = ps
        sq_ref[...] = psq

    @pl.when(step != 0)
    def _():
        sum_ref[...] += ps
        sq_ref[...] += psq


def _sc_stats_body(x_hbm, off_hbm, zeros_hbm, sum_out, sq_out,
                   buf0, buf1, off_v, acc_s, acc_q, sem0, sem1):
    c = lax.axis_index("c")
    s = lax.axis_index("s")
    wid = s * NC + c
    base = N_TC + wid * W_ROWS

    pltpu.sync_copy(zeros_hbm, acc_s)
    pltpu.sync_copy(zeros_hbm, acc_q)
    pltpu.sync_copy(off_hbm, off_v)

    off_vec = off_v[...]
    lane = lax.iota(jnp.int32, LANES)
    cl8 = [g * LANES + lane for g in range(8)]

    def start(ci, buf, sem):
        row0 = jnp.minimum(base + ci * CH, N - CH)
        pltpu.make_async_copy(x_hbm.at[pl.ds(row0, CH)], buf, sem).start()

    def wait(buf, sem):
        pltpu.make_async_copy(x_hbm.at[pl.ds(0, CH)], buf, sem).wait()

    def compute(ci, buf):
        row0 = base + ci * CH

        def seg_of(r):
            return plsc.all_reduce_population_count(
                off_vec <= jnp.full((LANES,), r, jnp.int32))

        def row_pair(jp, _):
            j0 = 2 * jp
            j1 = j0 + 1
            b0 = seg_of(row0 + j0) * (D // 128)
            b1 = seg_of(row0 + j1) * (D // 128)
            for k in range(D // LANES):
                sl = pl.ds(k * LANES, LANES)
                cl = cl8[k % 8]
                v0 = buf[j0, sl]
                v1 = buf[j1, sl]
                plsc.addupdate_scatter(acc_s, [b0 + (k // 8), cl], v0)
                plsc.addupdate_scatter(acc_s, [b1 + (k // 8), cl], v1)
                plsc.addupdate_scatter(acc_q, [b0 + (k // 8), cl], v0 * v0)
                plsc.addupdate_scatter(acc_q, [b1 + (k // 8), cl], v1 * v1)
            return 0
        lax.fori_loop(0, CH // 2, row_pair, 0)

    start(0, buf0, sem0)

    def pair(p, _):
        ci = 2 * p
        start(ci + 1, buf1, sem1)
        wait(buf0, sem0)
        compute(ci, buf0)
        start(ci + 2, buf0, sem0)
        wait(buf1, sem1)
        compute(ci + 1, buf1)
        return 0

    lax.fori_loop(0, W_ROWS // CH // 2, pair, 0)
    wait(buf0, sem0)

    rows_out = B * (D // 128)
    pltpu.sync_copy(acc_s, sum_out.at[pl.ds(wid * rows_out, rows_out)])
    pltpu.sync_copy(acc_q, sq_out.at[pl.ds(wid * rows_out, rows_out)])


def _sc_stats(x, offsets, interpret=False):
    zeros = jnp.zeros((B * (D // 128), 128), jnp.float32)
    fn = pl.kernel(
        _sc_stats_body,
        out_type=[jax.ShapeDtypeStruct((NW * B * (D // 128), 128),
                                       jnp.float32),
                  jax.ShapeDtypeStruct((NW * B * (D // 128), 128),
                                       jnp.float32)],
        mesh=plsc.VectorSubcoreMesh(core_axis_name="c",
                                    subcore_axis_name="s",
                                    num_cores=NC, num_subcores=NW // NC),
        scratch_types=[pltpu.VMEM((CH, D), jnp.float32),
                       pltpu.VMEM((CH, D), jnp.float32),
                       pltpu.VMEM((LANES,), jnp.int32),
                       pltpu.VMEM((B * (D // 128), 128), jnp.float32),
                       pltpu.VMEM((B * (D // 128), 128), jnp.float32),
                       pltpu.SemaphoreType.DMA,
                       pltpu.SemaphoreType.DMA],
        compiler_params=pltpu.CompilerParams(needs_layout_passes=False),
        interpret=interpret,
    )
    return fn(x, offsets, zeros)


def _prep_kernel(sum_ref, sq_ref, scs_ref, scq_ref, w_ref, b_ref, invc_ref,
                 scale_ref, shift_ref):
    inv = invc_ref[:, 0:1]
    s = sum_ref[...]
    sq = sq_ref[...]
    for w in range(NW):
        s = s + scs_ref[w * B:(w + 1) * B, :]
        sq = sq + scq_ref[w * B:(w + 1) * B, :]
    mean = s * inv
    var = sq * inv - mean * mean
    rstd = jax.lax.rsqrt(jnp.maximum(var, 0.0) + EPS)
    scale = rstd * w_ref[...]
    scale_ref[...] = scale
    shift_ref[...] = b_ref[...] - mean * scale


def _norm_kernel(x_ref, scale_ref, shift_ref, starts_ref, ends_ref, out_ref):
    step = pl.program_id(0)
    oh = _onehot(starts_ref, ends_ref, step, ROWS)
    dims = (((1,), (0,)), ((), ()))
    row_scale = jax.lax.dot_general(oh, scale_ref[...], dims,
                                    preferred_element_type=jnp.float32)
    row_shift = jax.lax.dot_general(oh, shift_ref[...], dims,
                                    preferred_element_type=jnp.float32)
    out_ref[...] = x_ref[...] * row_scale + row_shift


@functools.partial(jax.jit, static_argnames=("interpret",))
def kernel(input, offsets, weight, bias, interpret=False):
    s_steps = N_TC // S_ROWS
    steps = N // ROWS
    ends = offsets.reshape(1, B)
    starts = jnp.concatenate(
        [jnp.zeros((1, 1), jnp.int32), ends[:, :-1]], axis=1)
    invc = jnp.broadcast_to(
        (1.0 / jnp.maximum(ends - starts, 1).astype(jnp.float32)).reshape(
            B, 1), (B, 128))

    scs, scq = _sc_stats(input, offsets, interpret=interpret)
    scs = scs.reshape(NW * B, D)
    scq = scq.reshape(NW * B, D)

    small = pl.BlockSpec((1, B), lambda i: (0, 0))
    ssum, ssq = pl.pallas_call(
        _stats_kernel,
        grid=(s_steps,),
        in_specs=[pl.BlockSpec((S_ROWS, D), lambda i: (i, 0)), small, small],
        out_specs=[pl.BlockSpec((B, D), lambda i: (0, 0)),
                   pl.BlockSpec((B, D), lambda i: (0, 0))],
        out_shape=[jax.ShapeDtypeStruct((B, D), jnp.float32),
                   jax.ShapeDtypeStruct((B, D), jnp.float32)],
        interpret=interpret,
    )(input, starts, ends)

    scale, shift = pl.pallas_call(
        _prep_kernel,
        out_shape=[jax.ShapeDtypeStruct((B, D), jnp.float32),
                   jax.ShapeDtypeStruct((B, D), jnp.float32)],
        interpret=interpret,
    )(ssum, ssq, scs, scq, weight.reshape(1, D), bias.reshape(1, D), invc)

    out = pl.pallas_call(
        _norm_kernel,
        grid=(steps,),
        in_specs=[pl.BlockSpec((ROWS, D), lambda i: (i, 0)),
                  pl.BlockSpec((B, D), lambda i: (0, 0)),
                  pl.BlockSpec((B, D), lambda i: (0, 0)),
                  small, small],
        out_specs=pl.BlockSpec((ROWS, D), lambda i: (i, 0)),
        out_shape=jax.ShapeDtypeStruct((N, D), jnp.float32),
        interpret=interpret,
    )(input, scale, shift, starts, ends)
    return out


# speedup vs baseline: 1.2866x; 1.1437x over previous
import functools

import jax
import jax.numpy as jnp
from jax import lax
from jax.experimental import pallas as pl
from jax.experimental.pallas import tpu as pltpu
from jax.experimental.pallas import tpu_sc as plsc

N = 32768
B = 16
D = 1024
EPS = 1e-05

S_ROWS = 2048
ROWS = 2048

N_TC = 28672
N_SC = N - N_TC
NC = 2
NW = 32
W_ROWS = N_SC // NW
CH = 16
LANES = 16


def _onehot(starts_ref, ends_ref, step, rows):
    r = step * rows + jax.lax.broadcasted_iota(jnp.int32, (rows, B), 0)
    return ((r >= starts_ref[...]) & (r < ends_ref[...])).astype(jnp.float32)


def _stats_kernel(x_ref, starts_ref, ends_ref, sum_ref, sq_ref):
    step = pl.program_id(0)
    oh = _onehot(starts_ref, ends_ref, step, S_ROWS)
    x = x_ref[...]
    dims = (((0,), (0,)), ((), ()))
    ps = jax.lax.dot_general(oh, x, dims, preferred_element_type=jnp.float32)
    psq = jax.lax.dot_general(oh, x * x, dims,
                              preferred_element_type=jnp.float32)

    @pl.when(step == 0)
    def _():
        sum_ref[...] = ps
        sq_ref[...] = psq

    @pl.when(step != 0)
    def _():
        sum_ref[...] += ps
        sq_ref[...] += psq


def _sc_stats_body(x_hbm, off_hbm, zeros_hbm, sum_out, sq_out,
                   buf0, buf1, off_v, acc_s, acc_q, sem0, sem1):
    c = lax.axis_index("c")
    s = lax.axis_index("s")
    wid = s * NC + c
    base = N_TC + wid * W_ROWS

    pltpu.sync_copy(zeros_hbm, acc_s)
    pltpu.sync_copy(zeros_hbm, acc_q)
    pltpu.sync_copy(off_hbm, off_v)

    off_vec = off_v[...]
    lane = lax.iota(jnp.int32, LANES)
    cl8 = [g * LANES + lane for g in range(8)]

    def start(ci, buf, sem):
        row0 = jnp.minimum(base + ci * CH, N - CH)
        pltpu.make_async_copy(x_hbm.at[pl.ds(row0, CH)], buf, sem).start()

    def wait(buf, sem):
        pltpu.make_async_copy(x_hbm.at[pl.ds(0, CH)], buf, sem).wait()

    def compute(ci, buf):
        row0 = base + ci * CH

        def seg_of(r):
            return plsc.all_reduce_population_count(
                off_vec <= jnp.full((LANES,), r, jnp.int32))

        def row_pair(jp, _):
            j0 = 2 * jp
            j1 = j0 + 1
            b0 = seg_of(row0 + j0) * (D // 128)
            b1 = seg_of(row0 + j1) * (D // 128)
            for k in range(D // LANES):
                sl = pl.ds(k * LANES, LANES)
                cl = cl8[k % 8]
                v0 = buf[j0, sl]
                v1 = buf[j1, sl]
                plsc.addupdate_scatter(acc_s, [b0 + (k // 8), cl], v0)
                plsc.addupdate_scatter(acc_s, [b1 + (k // 8), cl], v1)
                plsc.addupdate_scatter(acc_q, [b0 + (k // 8), cl], v0 * v0)
                plsc.addupdate_scatter(acc_q, [b1 + (k // 8), cl], v1 * v1)
            return 0
        lax.fori_loop(0, CH // 2, row_pair, 0)

    start(0, buf0, sem0)

    def pair(p, _):
        ci = 2 * p
        start(ci + 1, buf1, sem1)
        wait(buf0, sem0)
        compute(ci, buf0)
        start(ci + 2, buf0, sem0)
        wait(buf1, sem1)
        compute(ci + 1, buf1)
        return 0

    lax.fori_loop(0, W_ROWS // CH // 2, pair, 0)
    wait(buf0, sem0)

    rows_out = B * (D // 128)
    pltpu.sync_copy(acc_s, sum_out.at[pl.ds(wid * rows_out, rows_out)])
    pltpu.sync_copy(acc_q, sq_out.at[pl.ds(wid * rows_out, rows_out)])


def _sc_stats(x, offsets, interpret=False):
    zeros = jnp.zeros((B * (D // 128), 128), jnp.float32)
    fn = pl.kernel(
        _sc_stats_body,
        out_type=[jax.ShapeDtypeStruct((NW * B * (D // 128), 128),
                                       jnp.float32),
                  jax.ShapeDtypeStruct((NW * B * (D // 128), 128),
                                       jnp.float32)],
        mesh=plsc.VectorSubcoreMesh(core_axis_name="c",
                                    subcore_axis_name="s",
                                    num_cores=NC, num_subcores=NW // NC),
        scratch_types=[pltpu.VMEM((CH, D), jnp.float32),
                       pltpu.VMEM((CH, D), jnp.float32),
                       pltpu.VMEM((LANES,), jnp.int32),
                       pltpu.VMEM((B * (D // 128), 128), jnp.float32),
                       pltpu.VMEM((B * (D // 128), 128), jnp.float32),
                       pltpu.SemaphoreType.DMA,
                       pltpu.SemaphoreType.DMA],
        compiler_params=pltpu.CompilerParams(needs_layout_passes=False),
        interpret=interpret,
    )
    return fn(x, offsets, zeros)


def _prep_kernel(sum_ref, sq_ref, scs_ref, scq_ref, w_ref, b_ref, invc_ref,
                 scale_ref, shift_ref):
    inv = invc_ref[:, 0:1]
    s = sum_ref[...]
    sq = sq_ref[...]
    for w in range(NW):
        s = s + scs_ref[w * B:(w + 1) * B, :]
        sq = sq + scq_ref[w * B:(w + 1) * B, :]
    mean = s * inv
    var = sq * inv - mean * mean
    rstd = jax.lax.rsqrt(jnp.maximum(var, 0.0) + EPS)
    scale = rstd * w_ref[...]
    scale_ref[...] = scale
    shift_ref[...] = b_ref[...] - mean * scale


def _norm_kernel(x_ref, scale_ref, shift_ref, starts_ref, ends_ref, out_ref):
    step = pl.program_id(0)
    oh = _onehot(starts_ref, ends_ref, step, ROWS)
    dims = (((1,), (0,)), ((), ()))
    row_scale = jax.lax.dot_general(oh, scale_ref[...], dims,
                                    preferred_element_type=jnp.float32)
    row_shift = jax.lax.dot_general(oh, shift_ref[...], dims,
                                    preferred_element_type=jnp.float32)
    out_ref[...] = x_ref[...] * row_scale + row_shift


@functools.partial(jax.jit, static_argnames=("interpret",))
def kernel(input, offsets, weight, bias, interpret=False):
    s_steps = N_TC // S_ROWS
    steps = N // ROWS
    ends = offsets.reshape(1, B)
    starts = jnp.concatenate(
        [jnp.zeros((1, 1), jnp.int32), ends[:, :-1]], axis=1)
    invc = jnp.broadcast_to(
        (1.0 / jnp.maximum(ends - starts, 1).astype(jnp.float32)).reshape(
            B, 1), (B, 128))

    scs, scq = _sc_stats(input, offsets, interpret=interpret)
    scs = scs.reshape(NW * B, D)
    scq = scq.reshape(NW * B, D)

    small = pl.BlockSpec((1, B), lambda i: (0, 0))
    ssum, ssq = pl.pallas_call(
        _stats_kernel,
        grid=(s_steps,),
        in_specs=[pl.BlockSpec((S_ROWS, D), lambda i: (i, 0)), small, small],
        out_specs=[pl.BlockSpec((B, D), lambda i: (0, 0)),
                   pl.BlockSpec((B, D), lambda i: (0, 0))],
        out_shape=[jax.ShapeDtypeStruct((B, D), jnp.float32),
                   jax.ShapeDtypeStruct((B, D), jnp.float32)],
        interpret=interpret,
    )(input, starts, ends)

    scale, shift = pl.pallas_call(
        _prep_kernel,
        out_shape=[jax.ShapeDtypeStruct((B, D), jnp.float32),
                   jax.ShapeDtypeStruct((B, D), jnp.float32)],
        interpret=interpret,
    )(ssum, ssq, scs, scq, weight.reshape(1, D), bias.reshape(1, D), invc)

    out = pl.pallas_call(
        _norm_kernel,
        grid=(steps,),
        in_specs=[pl.BlockSpec((ROWS, D), lambda i: (i, 0)),
                  pl.BlockSpec((B, D), lambda i: (0, 0)),
                  pl.BlockSpec((B, D), lambda i: (0, 0)),
                  small, small],
        out_specs=pl.BlockSpec((ROWS, D), lambda i: (i, 0)),
        out_shape=jax.ShapeDtypeStruct((N, D), jnp.float32),
        interpret=interpret,
    )(input, scale, shift, starts, ends)
    return out


# speedup vs baseline: 1.5873x; 1.2338x over previous
import functools

import jax
import jax.numpy as jnp
from jax.experimental import pallas as pl
from jax.experimental.pallas import tpu as pltpu

N = 32768
B = 16
D = 1024
EPS = 1e-05

ROWS = 2048


def _onehot(starts_ref, ends_ref, step, rows):
    r = step * rows + jax.lax.broadcasted_iota(jnp.int32, (rows, B), 0)
    return ((r >= starts_ref[...]) & (r < ends_ref[...])).astype(jnp.float32)


def _fused_kernel(x_ref, starts_ref, ends_ref, w_ref, b_ref, invc_ref,
                  out_ref, sum_s, sq_s, scale_s, shift_s):
    p = pl.program_id(0)
    i = pl.program_id(1)
    oh = _onehot(starts_ref, ends_ref, i, ROWS)

    @pl.when(p == 0)
    def _():
        x = x_ref[...]
        dims = (((0,), (0,)), ((), ()))
        ps = jax.lax.dot_general(oh, x, dims,
                                 preferred_element_type=jnp.float32)
        psq = jax.lax.dot_general(oh, x * x, dims,
                                  preferred_element_type=jnp.float32)

        @pl.when(i == 0)
        def _():
            sum_s[...] = ps
            sq_s[...] = psq

        @pl.when(i != 0)
        def _():
            sum_s[...] += ps
            sq_s[...] += psq

    @pl.when((p == 1) & (i == 0))
    def _():
        inv = invc_ref[:, 0:1]
        mean = sum_s[...] * inv
        var = sq_s[...] * inv - mean * mean
        rstd = jax.lax.rsqrt(jnp.maximum(var, 0.0) + EPS)
        scale = rstd * w_ref[...]
        scale_s[...] = scale
        shift_s[...] = b_ref[...] - mean * scale

    @pl.when(p == 1)
    def _():
        dims = (((1,), (0,)), ((), ()))
        row_scale = jax.lax.dot_general(oh, scale_s[...], dims,
                                        preferred_element_type=jnp.float32)
        row_shift = jax.lax.dot_general(oh, shift_s[...], dims,
                                        preferred_element_type=jnp.float32)
        out_ref[...] = x_ref[...] * row_scale + row_shift


@functools.partial(jax.jit, static_argnames=("interpret",))
def kernel(input, offsets, weight, bias, interpret=False):
    steps = N // ROWS
    ends = offsets.reshape(1, B)
    starts = jnp.concatenate(
        [jnp.zeros((1, 1), jnp.int32), ends[:, :-1]], axis=1)
    invc = jnp.broadcast_to(
        (1.0 / jnp.maximum(ends - starts, 1).astype(jnp.float32)).reshape(
            B, 1), (B, 128))

    small = pl.BlockSpec((1, B), lambda p, i: (0, 0))
    out = pl.pallas_call(
        _fused_kernel,
        grid=(2, steps),
        in_specs=[pl.BlockSpec((ROWS, D), lambda p, i: (i, 0)),
                  small, small,
                  pl.BlockSpec((1, D), lambda p, i: (0, 0)),
                  pl.BlockSpec((1, D), lambda p, i: (0, 0)),
                  pl.BlockSpec((B, 128), lambda p, i: (0, 0))],
        out_specs=pl.BlockSpec((ROWS, D), lambda p, i: (i * p, 0)),
        out_shape=jax.ShapeDtypeStruct((N, D), jnp.float32),
        scratch_shapes=[pltpu.VMEM((B, D), jnp.float32),
                        pltpu.VMEM((B, D), jnp.float32),
                        pltpu.VMEM((B, D), jnp.float32),
                        pltpu.VMEM((B, D), jnp.float32)],
        interpret=interpret,
    )(input, starts, ends, weight.reshape(1, D), bias.reshape(1, D), invc)
    return out
